# baseline (speedup 1.0000x reference)
import functools

import jax
import jax.numpy as jnp
from jax import lax
from jax.experimental import pallas as pl
from jax.experimental.pallas import tpu as pltpu
from jax.experimental.pallas import tpu_sc as plsc

VOCAB = 150528
HIDDEN = 12288
TOKENS = 8192

NC, NS = 2, 16
NW = NC * NS
ROWS_PER_W = TOKENS // NW
K = 2
NBUF = 4
CH = ROWS_PER_W // K
G = CH // NBUF

_mesh = plsc.VectorSubcoreMesh(
    core_axis_name="c", subcore_axis_name="s", num_cores=NC, num_subcores=NS
)


@functools.partial(
    pl.kernel,
    mesh=_mesh,
    out_type=jax.ShapeDtypeStruct((TOKENS, HIDDEN), jnp.float32),
    scratch_types=[
        pltpu.VMEM((CH, K), jnp.int32),
        [pltpu.VMEM((K, HIDDEN), jnp.float32) for _ in range(NBUF)],
        [pltpu.SemaphoreType.DMA for _ in range(NBUF)],
        [pltpu.SemaphoreType.DMA for _ in range(NBUF)],
    ],
)
def _sc_gather(idx_hbm, table_hbm, out_hbm, idx_v, bufs, gsem, wsem):
    wid = lax.axis_index("s") * NC + lax.axis_index("c")
    base = wid * ROWS_PER_W
    pltpu.sync_copy(idx_hbm.at[wid], idx_v)

    def gather_desc(c, b):
        return pltpu.make_async_copy(table_hbm.at[idx_v.at[c]], bufs[b], gsem[b])

    def write_desc(c, b):
        return pltpu.make_async_copy(
            bufs[b], out_hbm.at[pl.ds(base + c * K, K)], wsem[b]
        )

    for b in range(NBUF):
        gather_desc(b, b).start()

    def body(g, carry):
        for b in range(NBUF):
            c = NBUF * g + b
            gather_desc(c, b).wait()
            write_desc(c, b).start()
        for b in range(NBUF):
            c = NBUF * g + b
            write_desc(c, b).wait()
            gather_desc(c + NBUF, b).start()
        return carry

    lax.fori_loop(0, G - 1, body, 0)

    for b in range(NBUF):
        c = CH - NBUF + b
        gather_desc(c, b).wait()
        write_desc(c, b).start()
    for b in range(NBUF):
        write_desc(CH - NBUF + b, b).wait()


def kernel(input_ids, word_embeddings):
    b, s = input_ids.shape
    idx = input_ids.reshape(NW, CH, K)
    out = _sc_gather(idx, word_embeddings)
    return out.reshape(b, s, HIDDEN)
